# baseline (speedup 1.0000x reference)
import functools

import jax
import jax.numpy as jnp
from jax import lax
from jax.experimental import pallas as pl
from jax.experimental.pallas import tpu as pltpu
from jax.experimental.pallas import tpu_sc as plsc

N = 16384
EMB = 33
D_HALF = 32

NUM_CORES = 2
NUM_SUBCORES = 16
NW = NUM_CORES * NUM_SUBCORES
B_PER_W = N // NW
IDX_CHUNK = 128
N_CHUNKS = B_PER_W // IDX_CHUNK


_sc_mesh = plsc.VectorSubcoreMesh(core_axis_name="c", subcore_axis_name="s")


@functools.partial(
    pl.kernel,
    mesh=_sc_mesh,
    out_type=jax.ShapeDtypeStruct((N, D_HALF), jnp.float32),
    compiler_params=pltpu.CompilerParams(use_tc_tiling_on_sc=False),
    scratch_types=[
        pltpu.VMEM((N_CHUNKS, IDX_CHUNK), jnp.int32),
        pltpu.VMEM((B_PER_W, D_HALF), jnp.float32),
        pltpu.SemaphoreType.DMA,
    ],
)
def _gather_sc(table_hbm, idx_hbm, out_hbm, idx_v, rows_v, sem):
    wid = lax.axis_index("s") * NUM_CORES + lax.axis_index("c")
    base = wid * B_PER_W
    pltpu.sync_copy(idx_hbm.at[pl.ds(wid * N_CHUNKS, N_CHUNKS)], idx_v)
    copies = []
    for j in range(N_CHUNKS):
        copies.append(
            pltpu.async_copy(
                table_hbm.at[idx_v.at[j]],
                rows_v.at[pl.ds(j * IDX_CHUNK, IDX_CHUNK)],
                sem,
            )
        )
    for c in copies:
        c.wait()
    pltpu.sync_copy(rows_v, out_hbm.at[pl.ds(base, B_PER_W)])


def _proj_body(tok_ref, w_ref, b_ref, out_ref):
    x = tok_ref[:, 1:EMB]
    y = jnp.dot(x, w_ref[:].T, preferred_element_type=jnp.float32)
    out_ref[:] = jnp.maximum(y + b_ref[:], 0.0)


_PROJ_BLK = 2048


def _proj_tc(token, W, b):
    b2 = b.reshape(1, D_HALF)
    return pl.pallas_call(
        _proj_body,
        grid=(N // _PROJ_BLK,),
        in_specs=[
            pl.BlockSpec((_PROJ_BLK, EMB), lambda i: (i, 0)),
            pl.BlockSpec((D_HALF, EMB - 1), lambda i: (0, 0)),
            pl.BlockSpec((1, D_HALF), lambda i: (0, 0)),
        ],
        out_specs=pl.BlockSpec((_PROJ_BLK, D_HALF), lambda i: (i, 0)),
        out_shape=jax.ShapeDtypeStruct((N, D_HALF), jnp.float32),
    )(token, W, b2)


def kernel(token, table, W, b):
    idx = token[:, 0].astype(jnp.int32).reshape(NW * N_CHUNKS, IDX_CHUNK)
    emb = _gather_sc(table, idx)
    proj = _proj_tc(token, W, b)
    return jnp.concatenate([emb, proj], axis=1)
